# baseline (speedup 1.0000x reference)
import functools

import jax
import jax.numpy as jnp
from jax import lax
from jax.experimental import pallas as pl
from jax.experimental.pallas import tpu as pltpu
from jax.experimental.pallas import tpu_sc as plsc

N = 10000
E = 320000
D = 128
DH = D // 2

K = 125
ROWS = E // K
NC = 2
NS = 16
CPT_D = ROWS // (NC * NS)
CPT_E = ROWS // NS
DW = 16
WB = 624
TAIL = N - NS * WB
ZCH = 104
R = 8
HCH = CPT_E // 2

_MESH = plsc.VectorSubcoreMesh(core_axis_name="c", subcore_axis_name="s")


def _deg_body(dst_hbm, degw_hbm, dst_v, ones_v, zero_v, shared, sem):
    c = lax.axis_index("c")
    s = lax.axis_index("s")

    @pl.loop(0, WB)
    def _zfill(i):
        zero_v[i, :] = jnp.zeros((DW,), jnp.float32)

    @pl.loop(0, K)
    def _ofill(i):
        ones_v[i, :] = jnp.ones((DW,), jnp.float32)

    pltpu.sync_copy(zero_v, shared.at[pl.ds(s * WB, WB)])

    @pl.when(s == NS - 1)
    def _ztail():
        pltpu.sync_copy(zero_v.at[pl.ds(0, TAIL)],
                        shared.at[pl.ds(NS * WB, TAIL)])

    plsc.subcore_barrier()

    row0 = (c * NS + s) * CPT_D
    pltpu.sync_copy(dst_hbm.at[pl.ds(row0, CPT_D)], dst_v)

    @pl.loop(0, CPT_D // 4)
    def _scatter(g):
        sds = [
            pltpu.async_copy(ones_v, shared.at[dst_v.at[g * 4 + b]], sem,
                             add=True)
            for b in range(4)
        ]
        for d in sds:
            d.wait()

    plsc.subcore_barrier()
    pltpu.sync_copy(shared.at[pl.ds(s * WB, WB)],
                    degw_hbm.at[c, pl.ds(s * WB, WB)])

    @pl.when(s == NS - 1)
    def _wtail():
        pltpu.sync_copy(shared.at[pl.ds(NS * WB, TAIL)],
                        degw_hbm.at[c, pl.ds(NS * WB, TAIL)])


_deg = functools.partial(
    pl.kernel,
    out_type=jax.ShapeDtypeStruct((NC, N, DW), jnp.float32),
    mesh=_MESH,
    scratch_types=[
        pltpu.VMEM((CPT_D, K), jnp.int32),
        pltpu.VMEM((K, DW), jnp.float32),
        pltpu.VMEM((WB, DW), jnp.float32),
        pltpu.VMEM_SHARED((N, DW), jnp.float32),
        pltpu.SemaphoreType.DMA,
    ],
    compiler_params=pltpu.CompilerParams(use_tc_tiling_on_sc=False),
)(_deg_body)


def _edge_body(xws_hbm, src_hbm, dst_hbm, acc_hbm,
               src_v, dst_v, rowbuf, acc_shared, gsem, ssem):
    c = lax.axis_index("c")
    s = lax.axis_index("s")

    @pl.loop(0, ZCH)
    def _zfill(i):
        for k in range(DH // 16):
            rowbuf[i, pl.ds(k * 16, 16)] = jnp.zeros((16,), jnp.float32)

    zsrc = rowbuf.at[pl.ds(0, ZCH)]
    for i in range(WB // ZCH):
        pltpu.sync_copy(zsrc, acc_shared.at[pl.ds(s * WB + i * ZCH, ZCH)])

    @pl.when(s == NS - 1)
    def _ztail():
        pltpu.sync_copy(rowbuf.at[pl.ds(0, TAIL)],
                        acc_shared.at[pl.ds(NS * WB, TAIL)])

    plsc.subcore_barrier()

    row0 = s * CPT_E

    for h in range(2):
        pltpu.sync_copy(src_hbm.at[pl.ds(row0 + h * HCH, HCH)], src_v)
        pltpu.sync_copy(dst_hbm.at[pl.ds(row0 + h * HCH, HCH)], dst_v)

        @pl.loop(0, HCH // R)
        def _grp(g):
            j0 = g * R
            gds = [
                pltpu.async_copy(xws_hbm.at[c].at[src_v.at[j0 + b]],
                                 rowbuf.at[pl.ds(b * K, K)], gsem)
                for b in range(R)
            ]
            sds = []
            for b in range(R):
                gds[b].wait()
                sds.append(
                    pltpu.async_copy(rowbuf.at[pl.ds(b * K, K)],
                                     acc_shared.at[dst_v.at[j0 + b]], ssem,
                                     add=True))
            for d in sds:
                d.wait()

    plsc.subcore_barrier()
    pltpu.sync_copy(acc_shared.at[pl.ds(s * WB, WB)],
                    acc_hbm.at[c, pl.ds(s * WB, WB)])

    @pl.when(s == NS - 1)
    def _wtail():
        pltpu.sync_copy(acc_shared.at[pl.ds(NS * WB, TAIL)],
                        acc_hbm.at[c, pl.ds(NS * WB, TAIL)])


_edge_pass = functools.partial(
    pl.kernel,
    out_type=jax.ShapeDtypeStruct((NC, N, DH), jnp.float32),
    mesh=_MESH,
    scratch_types=[
        pltpu.VMEM((HCH, K), jnp.int32),
        pltpu.VMEM((HCH, K), jnp.int32),
        pltpu.VMEM((R * K, DH), jnp.float32),
        pltpu.VMEM_SHARED((N, DH), jnp.float32),
        pltpu.SemaphoreType.DMA,
        pltpu.SemaphoreType.DMA,
    ],
    compiler_params=pltpu.CompilerParams(use_tc_tiling_on_sc=False),
)(_edge_body)


ECH = 208


def _edge_fin_body(xws_hbm, src_hbm, dst_hbm, disw_hbm, b_hbm,
                   acc_hbm, out_hbm,
                   src_v, dst_v, rowbuf, dbuf, bbuf, acc_shared, gsem, ssem):
    c = lax.axis_index("c")
    s = lax.axis_index("s")

    @pl.loop(0, ZCH)
    def _zfill(i):
        for k in range(DH // 16):
            rowbuf[i, pl.ds(k * 16, 16)] = jnp.zeros((16,), jnp.float32)

    zsrc = rowbuf.at[pl.ds(0, ZCH)]
    for i in range(WB // ZCH):
        pltpu.sync_copy(zsrc, acc_shared.at[pl.ds(s * WB + i * ZCH, ZCH)])

    @pl.when(s == NS - 1)
    def _ztail():
        pltpu.sync_copy(rowbuf.at[pl.ds(0, TAIL)],
                        acc_shared.at[pl.ds(NS * WB, TAIL)])

    plsc.subcore_barrier()

    row0 = s * CPT_E
    for h in range(2):
        pltpu.sync_copy(src_hbm.at[pl.ds(row0 + h * HCH, HCH)], src_v)
        pltpu.sync_copy(dst_hbm.at[pl.ds(row0 + h * HCH, HCH)], dst_v)

        @pl.loop(0, HCH // R)
        def _grp(g):
            j0 = g * R
            gds = [
                pltpu.async_copy(xws_hbm.at[c].at[src_v.at[j0 + b]],
                                 rowbuf.at[pl.ds(b * K, K)], gsem)
                for b in range(R)
            ]
            sds = []
            for b in range(R):
                gds[b].wait()
                sds.append(
                    pltpu.async_copy(rowbuf.at[pl.ds(b * K, K)],
                                     acc_shared.at[dst_v.at[j0 + b]], ssem,
                                     add=True))
            for d in sds:
                d.wait()

    plsc.subcore_barrier()

    pltpu.sync_copy(acc_shared.at[pl.ds(s * WB, WB)],
                    acc_hbm.at[c, pl.ds(s * WB, WB)])

    @pl.when(s == NS - 1)
    def _wtail():
        pltpu.sync_copy(acc_shared.at[pl.ds(NS * WB, TAIL)],
                        acc_hbm.at[c, pl.ds(NS * WB, TAIL)])

    pltpu.sync_copy(b_hbm.at[c], bbuf)
    bv = [bbuf[pl.ds(k * 16, 16)] for k in range(DH // 16)]

    def _epilogue(r0, nr):
        d1 = pltpu.async_copy(acc_hbm.at[c, pl.ds(r0, nr)],
                              rowbuf.at[pl.ds(0, nr)], gsem)
        d2 = pltpu.async_copy(xws_hbm.at[c, pl.ds(r0, nr)],
                              rowbuf.at[pl.ds(ECH, nr)], gsem)
        d3 = pltpu.async_copy(disw_hbm.at[pl.ds(r0, nr)],
                              dbuf.at[pl.ds(0, nr)], gsem)
        d1.wait()
        d2.wait()
        d3.wait()

        @pl.loop(0, nr)
        def _rows(i):
            dv = dbuf[i, :]
            for k in range(DH // 16):
                sl = pl.ds(k * 16, 16)
                rowbuf[2 * ECH + i, sl] = (
                    (rowbuf[i, sl] + rowbuf[ECH + i, sl]) * dv + bv[k])

        pltpu.sync_copy(rowbuf.at[pl.ds(2 * ECH, nr)],
                        out_hbm.at[c, pl.ds(r0, nr)])

    for ch in range(WB // ECH):
        _epilogue(s * WB + ch * ECH, ECH)

    @pl.when(s == NS - 1)
    def _etail():
        _epilogue(NS * WB, TAIL)


_edge_fin = functools.partial(
    pl.kernel,
    out_type=(jax.ShapeDtypeStruct((NC, N, DH), jnp.float32),
              jax.ShapeDtypeStruct((NC, N, DH), jnp.float32)),
    mesh=_MESH,
    scratch_types=[
        pltpu.VMEM((HCH, K), jnp.int32),
        pltpu.VMEM((HCH, K), jnp.int32),
        pltpu.VMEM((R * K, DH), jnp.float32),
        pltpu.VMEM((ECH, DW), jnp.float32),
        pltpu.VMEM((DH,), jnp.float32),
        pltpu.VMEM_SHARED((N, DH), jnp.float32),
        pltpu.SemaphoreType.DMA,
        pltpu.SemaphoreType.DMA,
    ],
    compiler_params=pltpu.CompilerParams(use_tc_tiling_on_sc=False),
)(_edge_fin_body)


BR = 1000


def _dis(degw_ref):
    deg = degw_ref[0, :, 0:1] + degw_ref[1, :, 0:1] + 1.0
    return lax.rsqrt(deg)


def _halves(ref):
    return jnp.concatenate([ref[0], ref[1]], axis=1)


def _mm1_body(degw_ref, x_ref, w_ref, o_ref, disw_ref):
    dis = _dis(degw_ref)
    xw = jnp.dot(x_ref[...], w_ref[...], preferred_element_type=jnp.float32)
    xws = xw * dis
    o_ref[0] = xws[:, :DH]
    o_ref[1] = xws[:, DH:]
    disw_ref[...] = jnp.broadcast_to(dis, (BR, DW))


def _mm2_body(degw_ref, acc_ref, xws_ref, b_ref, w_ref, o_ref):
    dis = _dis(degw_ref)
    h = (_halves(acc_ref) + _halves(xws_ref)) * dis + b_ref[...]
    h = jnp.maximum(h, 0.01 * h)
    xws = jnp.dot(h, w_ref[...], preferred_element_type=jnp.float32) * dis
    o_ref[0] = xws[:, :DH]
    o_ref[1] = xws[:, DH:]


def _fin_body(degw_ref, acc_ref, xws_ref, b_ref, o_ref):
    dis = _dis(degw_ref)
    o_ref[...] = (_halves(acc_ref) + _halves(xws_ref)) * dis + b_ref[...]


_degw_spec = pl.BlockSpec((NC, BR, DW), lambda i: (0, i, 0))
_row_spec = pl.BlockSpec((BR, D), lambda i: (i, 0))
_half_spec = pl.BlockSpec((NC, BR, DH), lambda i: (0, i, 0))
_w_spec = pl.BlockSpec((D, D), lambda i: (0, 0))
_b_spec = pl.BlockSpec((1, D), lambda i: (0, 0))

_half_shape = jax.ShapeDtypeStruct((NC, N, DH), jnp.float32)

_mm1 = pl.pallas_call(
    _mm1_body,
    grid=(N // BR,),
    in_specs=[_degw_spec, _row_spec, _w_spec],
    out_specs=[_half_spec, pl.BlockSpec((BR, DW), lambda i: (i, 0))],
    out_shape=[_half_shape, jax.ShapeDtypeStruct((N, DW), jnp.float32)],
)

_mm2 = pl.pallas_call(
    _mm2_body,
    grid=(N // BR,),
    in_specs=[_degw_spec, _half_spec, _half_spec, _b_spec, _w_spec],
    out_specs=_half_spec,
    out_shape=_half_shape,
)

_fin = pl.pallas_call(
    _fin_body,
    grid=(N // BR,),
    in_specs=[_degw_spec, _half_spec, _half_spec, _b_spec],
    out_specs=_row_spec,
    out_shape=jax.ShapeDtypeStruct((N, D), jnp.float32),
)


@jax.jit
def kernel(x, edge_index, W1, b1, W2, b2):
    srcr = edge_index[0].reshape(ROWS, K)
    dstr = edge_index[1].reshape(ROWS, K)
    b1r = b1.reshape(1, D)
    b2r = b2.reshape(1, D)

    degw = _deg(dstr)
    xw1s, disw = _mm1(degw, x, W1)
    acc1 = _edge_pass(xw1s, srcr, dstr)
    xw2s = _mm2(degw, acc1, xw1s, b1r, W2)
    _, o = _edge_fin(xw2s, srcr, dstr, disw, b2.reshape(NC, DH))
    return jnp.concatenate([o[0], o[1]], axis=1)


# speedup vs baseline: 1.0189x; 1.0189x over previous
import functools

import jax
import jax.numpy as jnp
from jax import lax
from jax.experimental import pallas as pl
from jax.experimental.pallas import tpu as pltpu
from jax.experimental.pallas import tpu_sc as plsc

N = 10000
E = 320000
D = 128
DH = D // 2

K = 125
ROWS = E // K
NC = 2
NS = 16
CPT_D = ROWS // (NC * NS)
CPT_E = ROWS // NS
DW = 16
WB = 624
TAIL = N - NS * WB
ZCH = 104
R = 8
HCH = CPT_E // 2

_MESH = plsc.VectorSubcoreMesh(core_axis_name="c", subcore_axis_name="s")


def _deg_body(dst_hbm, degw_hbm, dst_v, ones_v, zero_v, shared, sem):
    c = lax.axis_index("c")
    s = lax.axis_index("s")

    @pl.loop(0, WB)
    def _zfill(i):
        zero_v[i, :] = jnp.zeros((DW,), jnp.float32)

    @pl.loop(0, K)
    def _ofill(i):
        ones_v[i, :] = jnp.ones((DW,), jnp.float32)

    pltpu.sync_copy(zero_v, shared.at[pl.ds(s * WB, WB)])

    @pl.when(s == NS - 1)
    def _ztail():
        pltpu.sync_copy(zero_v.at[pl.ds(0, TAIL)],
                        shared.at[pl.ds(NS * WB, TAIL)])

    plsc.subcore_barrier()

    row0 = (c * NS + s) * CPT_D
    pltpu.sync_copy(dst_hbm.at[pl.ds(row0, CPT_D)], dst_v)

    @pl.loop(0, CPT_D // 4)
    def _scatter(g):
        sds = [
            pltpu.async_copy(ones_v, shared.at[dst_v.at[g * 4 + b]], sem,
                             add=True)
            for b in range(4)
        ]
        for d in sds:
            d.wait()

    plsc.subcore_barrier()
    pltpu.sync_copy(shared.at[pl.ds(s * WB, WB)],
                    degw_hbm.at[c, pl.ds(s * WB, WB)])

    @pl.when(s == NS - 1)
    def _wtail():
        pltpu.sync_copy(shared.at[pl.ds(NS * WB, TAIL)],
                        degw_hbm.at[c, pl.ds(NS * WB, TAIL)])


_deg = functools.partial(
    pl.kernel,
    out_type=jax.ShapeDtypeStruct((NC, N, DW), jnp.float32),
    mesh=_MESH,
    scratch_types=[
        pltpu.VMEM((CPT_D, K), jnp.int32),
        pltpu.VMEM((K, DW), jnp.float32),
        pltpu.VMEM((WB, DW), jnp.float32),
        pltpu.VMEM_SHARED((N, DW), jnp.float32),
        pltpu.SemaphoreType.DMA,
    ],
    compiler_params=pltpu.CompilerParams(use_tc_tiling_on_sc=False),
)(_deg_body)


def _edge_body(xws_hbm, src_hbm, dst_hbm, acc_hbm,
               src_v, dst_v, rowbuf, acc_shared, gsem, ssem):
    c = lax.axis_index("c")
    s = lax.axis_index("s")

    @pl.loop(0, ZCH)
    def _zfill(i):
        for k in range(DH // 16):
            rowbuf[i, pl.ds(k * 16, 16)] = jnp.zeros((16,), jnp.float32)

    zsrc = rowbuf.at[pl.ds(0, ZCH)]
    for i in range(WB // ZCH):
        pltpu.sync_copy(zsrc, acc_shared.at[pl.ds(s * WB + i * ZCH, ZCH)])

    @pl.when(s == NS - 1)
    def _ztail():
        pltpu.sync_copy(rowbuf.at[pl.ds(0, TAIL)],
                        acc_shared.at[pl.ds(NS * WB, TAIL)])

    plsc.subcore_barrier()

    row0 = s * CPT_E

    for h in range(2):
        pltpu.sync_copy(src_hbm.at[pl.ds(row0 + h * HCH, HCH)], src_v)
        pltpu.sync_copy(dst_hbm.at[pl.ds(row0 + h * HCH, HCH)], dst_v)

        @pl.loop(0, HCH // R)
        def _grp(g):
            j0 = g * R
            gds = [
                pltpu.async_copy(xws_hbm.at[c].at[src_v.at[j0 + b]],
                                 rowbuf.at[pl.ds(b * K, K)], gsem)
                for b in range(R)
            ]
            sds = []
            for b in range(R):
                gds[b].wait()
                sds.append(
                    pltpu.async_copy(rowbuf.at[pl.ds(b * K, K)],
                                     acc_shared.at[dst_v.at[j0 + b]], ssem,
                                     add=True))
            for d in sds:
                d.wait()

    plsc.subcore_barrier()
    pltpu.sync_copy(acc_shared.at[pl.ds(s * WB, WB)],
                    acc_hbm.at[c, pl.ds(s * WB, WB)])

    @pl.when(s == NS - 1)
    def _wtail():
        pltpu.sync_copy(acc_shared.at[pl.ds(NS * WB, TAIL)],
                        acc_hbm.at[c, pl.ds(NS * WB, TAIL)])


_edge_pass = functools.partial(
    pl.kernel,
    out_type=jax.ShapeDtypeStruct((NC, N, DH), jnp.float32),
    mesh=_MESH,
    scratch_types=[
        pltpu.VMEM((HCH, K), jnp.int32),
        pltpu.VMEM((HCH, K), jnp.int32),
        pltpu.VMEM((R * K, DH), jnp.float32),
        pltpu.VMEM_SHARED((N, DH), jnp.float32),
        pltpu.SemaphoreType.DMA,
        pltpu.SemaphoreType.DMA,
    ],
    compiler_params=pltpu.CompilerParams(use_tc_tiling_on_sc=False),
)(_edge_body)


BR = 1000


def _dis(degw_ref):
    deg = degw_ref[0, :, 0:1] + degw_ref[1, :, 0:1] + 1.0
    return lax.rsqrt(deg)


def _halves(ref):
    return jnp.concatenate([ref[0], ref[1]], axis=1)


def _mm1_body(degw_ref, x_ref, w_ref, o_ref):
    dis = _dis(degw_ref)
    xw = jnp.dot(x_ref[...], w_ref[...], preferred_element_type=jnp.float32)
    xws = xw * dis
    o_ref[0] = xws[:, :DH]
    o_ref[1] = xws[:, DH:]


def _mm2_body(degw_ref, acc_ref, xws_ref, b_ref, w_ref, o_ref):
    dis = _dis(degw_ref)
    h = (_halves(acc_ref) + _halves(xws_ref)) * dis + b_ref[...]
    h = jnp.maximum(h, 0.01 * h)
    xws = jnp.dot(h, w_ref[...], preferred_element_type=jnp.float32) * dis
    o_ref[0] = xws[:, :DH]
    o_ref[1] = xws[:, DH:]


def _fin_body(degw_ref, acc_ref, xws_ref, b_ref, o_ref):
    dis = _dis(degw_ref)
    o_ref[...] = (_halves(acc_ref) + _halves(xws_ref)) * dis + b_ref[...]


_degw_spec = pl.BlockSpec((NC, BR, DW), lambda i: (0, i, 0))
_row_spec = pl.BlockSpec((BR, D), lambda i: (i, 0))
_half_spec = pl.BlockSpec((NC, BR, DH), lambda i: (0, i, 0))
_w_spec = pl.BlockSpec((D, D), lambda i: (0, 0))
_b_spec = pl.BlockSpec((1, D), lambda i: (0, 0))

_half_shape = jax.ShapeDtypeStruct((NC, N, DH), jnp.float32)

_mm1 = pl.pallas_call(
    _mm1_body,
    grid=(N // BR,),
    in_specs=[_degw_spec, _row_spec, _w_spec],
    out_specs=_half_spec,
    out_shape=_half_shape,
)

_mm2 = pl.pallas_call(
    _mm2_body,
    grid=(N // BR,),
    in_specs=[_degw_spec, _half_spec, _half_spec, _b_spec, _w_spec],
    out_specs=_half_spec,
    out_shape=_half_shape,
)

_fin = pl.pallas_call(
    _fin_body,
    grid=(N // BR,),
    in_specs=[_degw_spec, _half_spec, _half_spec, _b_spec],
    out_specs=_row_spec,
    out_shape=jax.ShapeDtypeStruct((N, D), jnp.float32),
)


@jax.jit
def kernel(x, edge_index, W1, b1, W2, b2):
    srcr = edge_index[0].reshape(ROWS, K)
    dstr = edge_index[1].reshape(ROWS, K)
    b1r = b1.reshape(1, D)
    b2r = b2.reshape(1, D)

    degw = _deg(dstr)
    xw1s = _mm1(degw, x, W1)
    acc1 = _edge_pass(xw1s, srcr, dstr)
    xw2s = _mm2(degw, acc1, xw1s, b1r, W2)
    acc2 = _edge_pass(xw2s, srcr, dstr)
    return _fin(degw, acc2, xw2s, b2r)


# speedup vs baseline: 1.0334x; 1.0142x over previous
import functools

import jax
import jax.numpy as jnp
from jax import lax
from jax.experimental import pallas as pl
from jax.experimental.pallas import tpu as pltpu
from jax.experimental.pallas import tpu_sc as plsc

N = 10000
E = 320000
D = 128
DH = D // 2

K = 125
ROWS = E // K
NC = 2
NS = 16
CPT_D = ROWS // (NC * NS)
CPT_E = ROWS // NS
DW = 16
WB = 624
TAIL = N - NS * WB
ZCH = 104
R = 8
HCH = CPT_E // 2

_MESH = plsc.VectorSubcoreMesh(core_axis_name="c", subcore_axis_name="s")


def _deg_body(dst_hbm, degw_hbm, dst_v, ones_v, zero_v, shared, sem):
    c = lax.axis_index("c")
    s = lax.axis_index("s")

    @pl.loop(0, WB)
    def _zfill(i):
        zero_v[i, :] = jnp.zeros((DW,), jnp.float32)

    @pl.loop(0, K)
    def _ofill(i):
        ones_v[i, :] = jnp.ones((DW,), jnp.float32)

    pltpu.sync_copy(zero_v, shared.at[pl.ds(s * WB, WB)])

    @pl.when(s == NS - 1)
    def _ztail():
        pltpu.sync_copy(zero_v.at[pl.ds(0, TAIL)],
                        shared.at[pl.ds(NS * WB, TAIL)])

    plsc.subcore_barrier()

    row0 = (c * NS + s) * CPT_D
    pltpu.sync_copy(dst_hbm.at[pl.ds(row0, CPT_D)], dst_v)

    @pl.loop(0, CPT_D // 4)
    def _scatter(g):
        sds = [
            pltpu.async_copy(ones_v, shared.at[dst_v.at[g * 4 + b]], sem,
                             add=True)
            for b in range(4)
        ]
        for d in sds:
            d.wait()

    plsc.subcore_barrier()
    pltpu.sync_copy(shared.at[pl.ds(s * WB, WB)],
                    degw_hbm.at[c, pl.ds(s * WB, WB)])

    @pl.when(s == NS - 1)
    def _wtail():
        pltpu.sync_copy(shared.at[pl.ds(NS * WB, TAIL)],
                        degw_hbm.at[c, pl.ds(NS * WB, TAIL)])


_deg = functools.partial(
    pl.kernel,
    out_type=jax.ShapeDtypeStruct((NC, N, DW), jnp.float32),
    mesh=_MESH,
    scratch_types=[
        pltpu.VMEM((CPT_D, K), jnp.int32),
        pltpu.VMEM((K, DW), jnp.float32),
        pltpu.VMEM((WB, DW), jnp.float32),
        pltpu.VMEM_SHARED((N, DW), jnp.float32),
        pltpu.SemaphoreType.DMA,
    ],
    compiler_params=pltpu.CompilerParams(use_tc_tiling_on_sc=False),
)(_deg_body)


def _edge_body(xws_hbm, src_hbm, dst_hbm, acc_hbm,
               src_v, dst_v, rowbuf, acc_shared, gsem, ssem):
    c = lax.axis_index("c")
    s = lax.axis_index("s")

    @pl.loop(0, ZCH)
    def _zfill(i):
        for k in range(DH // 16):
            rowbuf[i, pl.ds(k * 16, 16)] = jnp.zeros((16,), jnp.float32)

    zsrc = rowbuf.at[pl.ds(0, ZCH)]
    for i in range(WB // ZCH):
        pltpu.sync_copy(zsrc, acc_shared.at[pl.ds(s * WB + i * ZCH, ZCH)])

    @pl.when(s == NS - 1)
    def _ztail():
        pltpu.sync_copy(rowbuf.at[pl.ds(0, TAIL)],
                        acc_shared.at[pl.ds(NS * WB, TAIL)])

    plsc.subcore_barrier()

    row0 = s * CPT_E

    for h in range(2):
        pltpu.sync_copy(src_hbm.at[pl.ds(row0 + h * HCH, HCH)], src_v)
        pltpu.sync_copy(dst_hbm.at[pl.ds(row0 + h * HCH, HCH)], dst_v)

        @pl.loop(0, HCH // R)
        def _grp(g):
            j0 = g * R
            gds = [
                pltpu.async_copy(xws_hbm.at[c].at[src_v.at[j0 + b]],
                                 rowbuf.at[pl.ds(b * K, K)], gsem)
                for b in range(R)
            ]
            sds = []
            for b in range(R):
                gds[b].wait()
                sds.append(
                    pltpu.async_copy(rowbuf.at[pl.ds(b * K, K)],
                                     acc_shared.at[dst_v.at[j0 + b]], ssem,
                                     add=True))
            for d in sds:
                d.wait()

    plsc.subcore_barrier()
    pltpu.sync_copy(acc_shared.at[pl.ds(s * WB, WB)],
                    acc_hbm.at[c, pl.ds(s * WB, WB)])

    @pl.when(s == NS - 1)
    def _wtail():
        pltpu.sync_copy(acc_shared.at[pl.ds(NS * WB, TAIL)],
                        acc_hbm.at[c, pl.ds(NS * WB, TAIL)])


_edge_pass = functools.partial(
    pl.kernel,
    out_type=jax.ShapeDtypeStruct((NC, N, DH), jnp.float32),
    mesh=_MESH,
    scratch_types=[
        pltpu.VMEM((HCH, K), jnp.int32),
        pltpu.VMEM((HCH, K), jnp.int32),
        pltpu.VMEM((R * K, DH), jnp.float32),
        pltpu.VMEM_SHARED((N, DH), jnp.float32),
        pltpu.SemaphoreType.DMA,
        pltpu.SemaphoreType.DMA,
    ],
    compiler_params=pltpu.CompilerParams(use_tc_tiling_on_sc=False),
)(_edge_body)


BR = 2000


def _dis(degw_ref):
    deg = degw_ref[0, :, 0:1] + degw_ref[1, :, 0:1] + 1.0
    return lax.rsqrt(deg)


def _halves(ref):
    return jnp.concatenate([ref[0], ref[1]], axis=1)


def _mm1_body(degw_ref, x_ref, w_ref, o_ref):
    dis = _dis(degw_ref)
    xw = jnp.dot(x_ref[...], w_ref[...], preferred_element_type=jnp.float32)
    xws = xw * dis
    o_ref[0] = xws[:, :DH]
    o_ref[1] = xws[:, DH:]


def _mm2_body(degw_ref, acc_ref, xws_ref, b_ref, w_ref, o_ref):
    dis = _dis(degw_ref)
    h = (_halves(acc_ref) + _halves(xws_ref)) * dis + b_ref[...]
    h = jnp.maximum(h, 0.01 * h)
    xws = jnp.dot(h, w_ref[...], preferred_element_type=jnp.float32) * dis
    o_ref[0] = xws[:, :DH]
    o_ref[1] = xws[:, DH:]


def _fin_body(degw_ref, acc_ref, xws_ref, b_ref, o_ref):
    dis = _dis(degw_ref)
    o_ref[...] = (_halves(acc_ref) + _halves(xws_ref)) * dis + b_ref[...]


_degw_spec = pl.BlockSpec((NC, BR, DW), lambda i: (0, i, 0))
_row_spec = pl.BlockSpec((BR, D), lambda i: (i, 0))
_half_spec = pl.BlockSpec((NC, BR, DH), lambda i: (0, i, 0))
_w_spec = pl.BlockSpec((D, D), lambda i: (0, 0))
_b_spec = pl.BlockSpec((1, D), lambda i: (0, 0))

_half_shape = jax.ShapeDtypeStruct((NC, N, DH), jnp.float32)

_mm1 = pl.pallas_call(
    _mm1_body,
    grid=(N // BR,),
    in_specs=[_degw_spec, _row_spec, _w_spec],
    out_specs=_half_spec,
    out_shape=_half_shape,
)

_mm2 = pl.pallas_call(
    _mm2_body,
    grid=(N // BR,),
    in_specs=[_degw_spec, _half_spec, _half_spec, _b_spec, _w_spec],
    out_specs=_half_spec,
    out_shape=_half_shape,
)

_fin = pl.pallas_call(
    _fin_body,
    grid=(N // BR,),
    in_specs=[_degw_spec, _half_spec, _half_spec, _b_spec],
    out_specs=_row_spec,
    out_shape=jax.ShapeDtypeStruct((N, D), jnp.float32),
)


@jax.jit
def kernel(x, edge_index, W1, b1, W2, b2):
    srcr = edge_index[0].reshape(ROWS, K)
    dstr = edge_index[1].reshape(ROWS, K)
    b1r = b1.reshape(1, D)
    b2r = b2.reshape(1, D)

    degw = _deg(dstr)
    xw1s = _mm1(degw, x, W1)
    acc1 = _edge_pass(xw1s, srcr, dstr)
    xw2s = _mm2(degw, acc1, xw1s, b1r, W2)
    acc2 = _edge_pass(xw2s, srcr, dstr)
    return _fin(degw, acc2, xw2s, b2r)
